# baseline (speedup 1.0000x reference)
import functools

import jax
import jax.numpy as jnp
from jax import lax
from jax.experimental import pallas as pl
from jax.experimental.pallas import tpu as pltpu
from jax.experimental.pallas import tpu_sc as plsc

NUM_CODES = 8192
DIM = 256
B_TOTAL = 16384
BETA = 0.25

BLOCK_B = 2048
N_BLOCKS = B_TOTAL // BLOCK_B

NUM_SC = 2
NUM_SUBCORES = 16
NW = NUM_SC * NUM_SUBCORES
ROWS_PER_W = B_TOTAL // NW
CHUNK = 128


def _dist_body(z_ref, cb_ref, idx_ref, loss_ref, e2_ref, cbh_ref, acc_ref):
    i = pl.program_id(0)

    @pl.when(i == 0)
    def _():
        acc_ref[0, 0] = jnp.float32(0.0)
        cb = cb_ref[...]
        e2_ref[...] = lax.dot_general(
            jnp.ones((1, DIM), jnp.float32), cb * cb, (((1,), (1,)), ((), ())),
            preferred_element_type=jnp.float32,
            precision=lax.Precision.DEFAULT)
        cbh_ref[...] = cb.astype(jnp.bfloat16)

    z = z_ref[...]
    z2 = jnp.sum(z * z, axis=1, keepdims=True)
    zd = z + z
    zh = zd.astype(jnp.bfloat16)
    e2 = e2_ref[...]

    TILE = 128
    RC = 64
    NRC = BLOCK_B // RC
    CHUNK_K = 1024
    NCH = NUM_CODES // CHUNK_K
    TPC = CHUNK_K // TILE
    dn = (((1,), (1,)), ((), ()))
    z2c = [z2[rc * RC:(rc + 1) * RC, :] for rc in range(NRC)]
    run_min = [jnp.full((RC, TILE), jnp.inf, jnp.float32) for _ in range(NRC)]
    run_tile = [jnp.zeros((RC, TILE), jnp.int32) for _ in range(NRC)]
    for c in range(NCH):
        ch = cbh_ref[c * CHUNK_K:(c + 1) * CHUNK_K, :]
        ze2 = lax.dot_general(zh, ch, dn, preferred_element_type=jnp.float32)
        for rc in range(NRC):
            for tc in range(TPC):
                t = c * TPC + tc
                d = ((z2c[rc] + e2[:, t * TILE:(t + 1) * TILE])
                     - ze2[rc * RC:(rc + 1) * RC, tc * TILE:(tc + 1) * TILE])
                m = d < run_min[rc]
                run_min[rc] = jnp.minimum(d, run_min[rc])
                run_tile[rc] = jnp.where(m, t, run_tile[rc])
    loss = jnp.float32(0.0)
    for rc in range(NRC):
        mrow = jnp.min(run_min[rc], axis=1, keepdims=True)
        lane = lax.broadcasted_iota(jnp.int32, (RC, TILE), 1)
        gidx = run_tile[rc] * TILE + lane
        idx = jnp.min(jnp.where(run_min[rc] == mrow, gidx, NUM_CODES), axis=1)
        idx_ref[0, 0, rc * RC:(rc + 1) * RC] = idx
        loss = loss + jnp.sum(mrow[:, 0])
    acc_ref[0, 0] = acc_ref[0, 0] + loss

    @pl.when(i == N_BLOCKS - 1)
    def _():
        cb_loss = acc_ref[0, 0] / (B_TOTAL * DIM)
        loss_ref[0, 0, 0] = cb_loss + BETA * cb_loss
        loss_ref[0, 0, 1] = cb_loss
        loss_ref[0, 0, 2] = cb_loss


def _build_dist_call(interpret=False):
    return pl.pallas_call(
        _dist_body,
        grid=(N_BLOCKS,),
        in_specs=[
            pl.BlockSpec((BLOCK_B, DIM), lambda i: (i, 0)),
            pl.BlockSpec((NUM_CODES, DIM), lambda i: (0, 0)),
        ],
        out_specs=[
            pl.BlockSpec((1, 1, BLOCK_B), lambda i: (i, 0, 0)),
            pl.BlockSpec((1, 1, 4), lambda i: (0, 0, 0), memory_space=pltpu.SMEM),
        ],
        out_shape=[
            jax.ShapeDtypeStruct((N_BLOCKS, 1, BLOCK_B), jnp.int32),
            jax.ShapeDtypeStruct((1, 1, 4), jnp.float32),
        ],
        scratch_shapes=[
            pltpu.VMEM((1, NUM_CODES), jnp.float32),
            pltpu.VMEM((NUM_CODES, DIM), jnp.bfloat16),
            pltpu.SMEM((1, 1), jnp.float32),
        ],
        interpret=interpret,
    )


_dist_call = _build_dist_call()


def _gather_body(cb_hbm, idx_hbm, out_hbm,
                 idx0, idx1, rows0, rows1, gsem0, gsem1, ssem0, ssem1):
    wid = lax.axis_index("s") * NUM_SC + lax.axis_index("c")
    base = wid * ROWS_PER_W
    idxs = [idx0, idx1]
    rows = [rows0, rows1]
    gsems = [gsem0, gsem1]
    ssems = [ssem0, ssem1]
    nchunk = ROWS_PER_W // CHUNK
    stores = [None] * nchunk
    for c in range(nchunk):
        b = c % 2
        if c >= 2:
            stores[c - 2].wait()
        off = base + c * CHUNK
        pltpu.sync_copy(idx_hbm.at[pl.ds(off, CHUNK)], idxs[b])
        pltpu.async_copy(cb_hbm.at[idxs[b]], rows[b], gsems[b]).wait()
        stores[c] = pltpu.async_copy(rows[b], out_hbm.at[pl.ds(off, CHUNK)],
                                     ssems[b])
    for c in range(nchunk - 2, nchunk):
        stores[c].wait()


@functools.cache
def _build_gather_call():
    return pl.kernel(
        _gather_body,
        out_type=jax.ShapeDtypeStruct((B_TOTAL, DIM), jnp.float32),
        mesh=plsc.VectorSubcoreMesh(core_axis_name="c", subcore_axis_name="s"),
        scratch_types=[
            pltpu.VMEM((CHUNK,), jnp.int32),
            pltpu.VMEM((CHUNK,), jnp.int32),
            pltpu.VMEM((CHUNK, DIM), jnp.float32),
            pltpu.VMEM((CHUNK, DIM), jnp.float32),
            pltpu.SemaphoreType.DMA,
            pltpu.SemaphoreType.DMA,
            pltpu.SemaphoreType.DMA,
            pltpu.SemaphoreType.DMA,
        ],
    )


def kernel(z_e, codebook):
    idx3, losses = _dist_call(z_e, codebook)
    indices = idx3.reshape(B_TOTAL)
    z_q_st = _build_gather_call()(codebook, indices)
    vq_loss = losses[0, 0, 0]
    cb_loss = losses[0, 0, 1]
    commit_loss = losses[0, 0, 2]
    return (z_q_st, indices, vq_loss, cb_loss, commit_loss)
